# Initial kernel scaffold; baseline (speedup 1.0000x reference)
import jax
import jax.numpy as jnp
from jax.experimental import pallas as pl


def kernel(x, batch, W1, b1, W2, b2, W3, b3):
    raise NotImplementedError("write your pallas kernel here")


# speedup vs baseline: 8.2213x; 8.2213x over previous
import jax
import jax.numpy as jnp
from jax.experimental import pallas as pl
from jax.experimental.pallas import tpu as pltpu

N_NODES = 100000
D_FEAT = 128
NUM_GRAPHS = 256
N_CLASSES = 4
BLK = 2000


def _fused_body(x_ref, ids_ref, w1t_ref, b1_ref, w2t_ref, b2_ref,
                w3t_ref, b3_ref, out_ref, acc_ref, cnt_ref):
    i = pl.program_id(0)
    nsteps = pl.num_programs(0)

    @pl.when(i == 0)
    def _init():
        acc_ref[...] = jnp.zeros_like(acc_ref)
        cnt_ref[...] = jnp.zeros_like(cnt_ref)

    h = jnp.maximum(
        jnp.dot(x_ref[...], w1t_ref[...], preferred_element_type=jnp.float32)
        + b1_ref[...], 0.0)
    h = jnp.maximum(
        jnp.dot(h, w2t_ref[...], preferred_element_type=jnp.float32)
        + b2_ref[...], 0.0)

    ids = ids_ref[0]
    seg_iota = jax.lax.broadcasted_iota(jnp.int32, (NUM_GRAPHS, BLK), 0)
    oh_t = (seg_iota == ids).astype(jnp.float32)
    acc_ref[...] += jax.lax.dot_general(
        oh_t, h, (((1,), (0,)), ((), ())),
        preferred_element_type=jnp.float32)
    cnt_ref[...] += jnp.sum(oh_t, axis=1, keepdims=True)

    @pl.when(i == nsteps - 1)
    def _head():
        pooled = acc_ref[...] / jnp.maximum(cnt_ref[...], 1.0)
        out_ref[...] = (
            jnp.dot(pooled, w3t_ref[...], preferred_element_type=jnp.float32)
            + b3_ref[...])


def kernel(x, batch, W1, b1, W2, b2, W3, b3):
    nblk = N_NODES // BLK
    ids3d = batch.astype(jnp.int32).reshape(nblk, 1, BLK)
    grid = (nblk,)
    out = pl.pallas_call(
        _fused_body,
        grid=grid,
        in_specs=[
            pl.BlockSpec((BLK, D_FEAT), lambda i: (i, 0)),
            pl.BlockSpec((1, 1, BLK), lambda i: (i, 0, 0)),
            pl.BlockSpec((D_FEAT, D_FEAT), lambda i: (0, 0)),
            pl.BlockSpec((1, D_FEAT), lambda i: (0, 0)),
            pl.BlockSpec((D_FEAT, D_FEAT), lambda i: (0, 0)),
            pl.BlockSpec((1, D_FEAT), lambda i: (0, 0)),
            pl.BlockSpec((D_FEAT, N_CLASSES), lambda i: (0, 0)),
            pl.BlockSpec((1, N_CLASSES), lambda i: (0, 0)),
        ],
        out_specs=pl.BlockSpec((NUM_GRAPHS, N_CLASSES), lambda i: (0, 0)),
        out_shape=jax.ShapeDtypeStruct((NUM_GRAPHS, N_CLASSES), jnp.float32),
        scratch_shapes=[
            pltpu.VMEM((NUM_GRAPHS, D_FEAT), jnp.float32),
            pltpu.VMEM((NUM_GRAPHS, 1), jnp.float32),
        ],
        compiler_params=pltpu.CompilerParams(
            dimension_semantics=("arbitrary",)),
    )(x, ids3d, W1.T, b1.reshape(1, D_FEAT), W2.T, b2.reshape(1, D_FEAT),
      W3.T, b3.reshape(1, N_CLASSES))
    return out
